# Initial kernel scaffold; baseline (speedup 1.0000x reference)
import jax
import jax.numpy as jnp
from jax.experimental import pallas as pl


def kernel(feat, edge_index, eps):
    raise NotImplementedError("write your pallas kernel here")


# speedup vs baseline: 3.8454x; 3.8454x over previous
import functools

import jax
import jax.numpy as jnp
from jax import lax
from jax.experimental import pallas as pl
from jax.experimental.pallas import tpu as pltpu
from jax.experimental.pallas import tpu_sc as plsc

N = 10000
E = 320000
D = 128

NC = 2
NS = 16
NW = NC * NS

K = 128
E_PER_W = 10240
CHUNKS = E_PER_W // K
E_PAD = NW * E_PER_W
N_PAD = 10240
STRIPE = N_PAD // NS
LANES = 16


def _sc_aggregate(feat_pad, src_r, dst_r, zblk, ones, zvec):
    mesh = plsc.VectorSubcoreMesh(core_axis_name="c", subcore_axis_name="s")

    @functools.partial(
        pl.kernel,
        mesh=mesh,
        out_type=jax.ShapeDtypeStruct((NC, N_PAD, D), jnp.float32),
        scratch_types=[
            pltpu.VMEM((CHUNKS, K), jnp.int32),
            pltpu.VMEM((2 * CHUNKS, K), jnp.int32),
            pltpu.VMEM((K, D), jnp.float32),
            pltpu.VMEM((K,), jnp.float32),
            pltpu.VMEM((STRIPE,), jnp.float32),
            pltpu.VMEM_SHARED((N_PAD, D), jnp.float32),
            pltpu.VMEM_SHARED((N_PAD,), jnp.float32),
        ],
    )
    def agg(feat_hbm, src_hbm, dst_hbm, zblk_hbm, ones_hbm, zvec_hbm, q_hbm,
            src_v, dst_v, rows_v, ones_v, dstripe_v, accum_sh, deg_sh):
        c = lax.axis_index("c")
        s = lax.axis_index("s")
        wid = c * NS + s

        pltpu.sync_copy(zblk_hbm, rows_v)
        pltpu.sync_copy(ones_hbm, ones_v)
        pltpu.sync_copy(zvec_hbm, dstripe_v)
        pltpu.sync_copy(src_hbm.at[wid], src_v)
        pltpu.sync_copy(dst_hbm.at[s], dst_v.at[pl.ds(0, CHUNKS)])
        pltpu.sync_copy(dst_hbm.at[NS + s], dst_v.at[pl.ds(CHUNKS, CHUNKS)])

        base = s * STRIPE
        for b in range(STRIPE // K):
            pltpu.sync_copy(rows_v, accum_sh.at[pl.ds(base + b * K, K)])
        pltpu.sync_copy(dstripe_v, deg_sh.at[pl.ds(base, STRIPE)])
        plsc.subcore_barrier()

        own = c * CHUNKS

        def body(j, carry):
            pltpu.sync_copy(ones_v, deg_sh.at[dst_v.at[j]], add=True)
            pltpu.sync_copy(ones_v, deg_sh.at[dst_v.at[CHUNKS + j]], add=True)
            pltpu.sync_copy(feat_hbm.at[src_v.at[j]], rows_v)
            pltpu.sync_copy(rows_v, accum_sh.at[dst_v.at[own + j]], add=True)
            return carry

        lax.fori_loop(0, CHUNKS, body, 0)
        plsc.subcore_barrier()

        pltpu.sync_copy(deg_sh.at[pl.ds(base, STRIPE)], dstripe_v)

        def inv_body(t, carry):
            sl = pl.ds(t * LANES, LANES)
            dstripe_v[sl] = 1.0 / jnp.maximum(dstripe_v[sl], 1.0)
            return carry

        lax.fori_loop(0, STRIPE // LANES, inv_body, 0)

        for b in range(STRIPE // K):
            pltpu.sync_copy(accum_sh.at[pl.ds(base + b * K, K)], rows_v)

            def div_row(r, carry):
                g = (r // LANES) * LANES
                dv = dstripe_v[pl.ds(b * K + g, LANES)]
                lane = jnp.full((LANES,), r - g, jnp.int32)
                invv = dv.at[lane].get(mode="promise_in_bounds")
                for v in range(D // LANES):
                    sl = pl.ds(v * LANES, LANES)
                    rows_v[r, sl] = rows_v[r, sl] * invv
                return carry

            lax.fori_loop(0, K, div_row, 0)
            pltpu.sync_copy(rows_v, q_hbm.at[c, pl.ds(base + b * K, K)])

    return agg(feat_pad, src_r, dst_r, zblk, ones, zvec)


def kernel(feat, edge_index, eps):
    src = edge_index[0]
    dst = edge_index[1]
    pad = E_PAD - E
    src_r = jnp.concatenate([src, jnp.zeros((pad,), jnp.int32)]).reshape(NW, CHUNKS, K)
    dst_r = jnp.concatenate([dst, jnp.full((pad,), N, jnp.int32)]).reshape(NW, CHUNKS, K)
    feat_pad = jnp.concatenate(
        [feat, jnp.zeros((N_PAD - N, D), jnp.float32)], axis=0)

    q = _sc_aggregate(feat_pad, src_r, dst_r,
                      jnp.zeros((K, D), jnp.float32),
                      jnp.ones((K,), jnp.float32),
                      jnp.zeros((STRIPE,), jnp.float32))

    BLK = 512
    eps2 = jnp.reshape(eps, (1, 1)).astype(jnp.float32)

    def combine(eps_ref, feat_ref, q0_ref, q1_ref, out_ref):
        out_ref[...] = ((1.0 + eps_ref[0, 0]) * feat_ref[...]
                        + q0_ref[0] + q1_ref[0])

    out = pl.pallas_call(
        combine,
        grid=(N_PAD // BLK,),
        in_specs=[
            pl.BlockSpec((1, 1), lambda i: (0, 0)),
            pl.BlockSpec((BLK, D), lambda i: (i, 0)),
            pl.BlockSpec((1, BLK, D), lambda i: (0, i, 0)),
            pl.BlockSpec((1, BLK, D), lambda i: (1, i, 0)),
        ],
        out_specs=pl.BlockSpec((BLK, D), lambda i: (i, 0)),
        out_shape=jax.ShapeDtypeStruct((N_PAD, D), jnp.float32),
    )(eps2, feat_pad, q, q)
    return out[:N]
